# Initial kernel scaffold; baseline (speedup 1.0000x reference)
import functools

import jax
import jax.numpy as jnp
import numpy as np
from jax import lax
from jax.experimental import pallas as pl

_GRID = 32
_D = 96
_N_S = _GRID * _GRID
_ND = _N_S // 2
_NSRC = _N_S // 2
_KEEP_RATE = 0.7
_ALPHA = 0.7
_R = int(_N_S * (1.0 - _KEEP_RATE))
_NUNM = _NSRC - _R
_NOUT = 1 + _ND + _NUNM
_HI = lax.Precision.HIGHEST


def _build_low_basis():
    h = w = _GRID
    y, x = np.meshgrid(np.arange(h), np.arange(w), indexing="ij")
    cols = [np.full((h, w), 1.0 / np.sqrt(h * w))]
    for dy, dx in [(0, 1), (0, 2), (1, 0), (2, 0), (1, 1), (1, -1)]:
        ph = 2.0 * np.pi * (dy * y + dx * x) / h
        cols.append(np.sqrt(2.0 / (h * w)) * np.cos(ph))
        cols.append(np.sqrt(2.0 / (h * w)) * np.sin(ph))
    u = np.stack([c.reshape(-1) for c in cols], axis=1)
    u = np.concatenate([u, np.zeros((h * w, 3))], axis=1)
    return u.astype(np.float32)


_U = _build_low_basis()
_UD = jnp.asarray(_U[0::2])
_US = jnp.asarray(_U[1::2])


def _dot(a, b, ca, cb):
    return lax.dot_general(a, b, (((ca,), (cb,)), ((), ())),
                           preferred_element_type=jnp.float32, precision=_HI)


def _body(cls_ref, dst_ref, src_ref, ud_ref, us_ref, out_ref):
    xd = dst_ref[0]
    xs = src_ref[0]
    ud = ud_ref[...]
    us = us_ref[...]

    coef = _dot(ud, xd, 0, 0) + _dot(us, xs, 0, 0)
    hfd = xd - _dot(ud, coef, 1, 0)
    hfs = xs - _dot(us, coef, 1, 0)
    ed = jnp.sqrt(jnp.sum(hfd * hfd, axis=1))
    es = jnp.sqrt(jnp.sum(hfs * hfs, axis=1))
    pmin = jnp.minimum(jnp.min(ed), jnp.min(es))
    pmax = jnp.maximum(jnp.max(ed), jnp.max(es))
    inv = 1.0 / (pmax - pmin + 1e-6)
    phid = (ed - pmin) * inv
    phis = (es - pmin) * inv

    nd = xd / jnp.maximum(jnp.sqrt(jnp.sum(xd * xd, axis=1)), 1e-12)[:, None]
    ns = xs / jnp.maximum(jnp.sqrt(jnp.sum(xs * xs, axis=1)), 1e-12)[:, None]
    sim = _dot(ns, nd, 1, 1)
    adj = sim * (1.0 - _ALPHA * jnp.maximum(phis[:, None], phid[None, :]))

    jcol = lax.broadcasted_iota(jnp.int32, (_NSRC, _ND), 1)
    node_max = jnp.max(adj, axis=1)
    node_idx = jnp.min(jnp.where(adj == node_max[:, None], jcol, _ND), axis=1)

    irow = lax.broadcasted_iota(jnp.int32, (_NSRC, _NSRC), 0)
    vi = node_max[:, None]
    vj = node_max[None, :]
    better = (vj > vi) | ((vj == vi) & (jcol < irow))
    rank = jnp.sum(better.astype(jnp.float32), axis=1)
    merged = rank < (_R - 0.5)
    mf = merged.astype(jnp.float32)

    sel = jnp.where(node_idx[:, None] == jcol, mf[:, None], 0.0)
    addv = _dot(sel, xs, 0, 0)
    cnt = jnp.sum(sel, axis=0)
    dst_out = (xd + addv) / (1.0 + cnt)[:, None]

    keep = 1.0 - mf
    pos = jnp.sum(jnp.where(jcol < irow, keep[None, :], 0.0), axis=1)
    prow = lax.broadcasted_iota(jnp.float32, (_NUNM, _NSRC), 0)
    gat = jnp.where((prow == pos[None, :]) & (keep[None, :] > 0.5), 1.0, 0.0)
    unm = _dot(gat, xs, 1, 0)

    out_ref[0] = jnp.concatenate([cls_ref[0], dst_out, unm], axis=0)


@functools.partial(jax.jit, static_argnums=())
def kernel(tokens):
    b = tokens.shape[0]
    cls_tok = tokens[:, :1]
    dst = tokens[:, 1::2]
    src = tokens[:, 2::2]
    return pl.pallas_call(
        _body,
        grid=(b,),
        in_specs=[
            pl.BlockSpec((1, 1, _D), lambda i: (i, 0, 0)),
            pl.BlockSpec((1, _ND, _D), lambda i: (i, 0, 0)),
            pl.BlockSpec((1, _NSRC, _D), lambda i: (i, 0, 0)),
            pl.BlockSpec((_ND, 16), lambda i: (0, 0)),
            pl.BlockSpec((_NSRC, 16), lambda i: (0, 0)),
        ],
        out_specs=pl.BlockSpec((1, _NOUT, _D), lambda i: (i, 0, 0)),
        out_shape=jax.ShapeDtypeStruct((b, _NOUT, _D), jnp.float32),
    )(cls_tok, dst, src, _UD, _US)


# speedup vs baseline: 6.7777x; 6.7777x over previous
import functools

import jax
import jax.numpy as jnp
import numpy as np
from jax import lax
from jax.experimental import pallas as pl

_GRID = 32
_D = 96
_N_S = _GRID * _GRID
_ND = _N_S // 2
_NSRC = _N_S // 2
_KEEP_RATE = 0.7
_ALPHA = 0.7
_R = int(_N_S * (1.0 - _KEEP_RATE))
_NUNM = _NSRC - _R
_NOUT = 1 + _ND + _NUNM
_HI = lax.Precision.HIGHEST


def _build_low_basis():
    h = w = _GRID
    y, x = np.meshgrid(np.arange(h), np.arange(w), indexing="ij")
    cols = [np.full((h, w), 1.0 / np.sqrt(h * w))]
    for dy, dx in [(0, 1), (0, 2), (1, 0), (2, 0), (1, 1), (1, -1)]:
        ph = 2.0 * np.pi * (dy * y + dx * x) / h
        cols.append(np.sqrt(2.0 / (h * w)) * np.cos(ph))
        cols.append(np.sqrt(2.0 / (h * w)) * np.sin(ph))
    u = np.stack([c.reshape(-1) for c in cols], axis=1)
    u = np.concatenate([u, np.zeros((h * w, 3))], axis=1)
    return u.astype(np.float32)


_U = _build_low_basis()
_UD = jnp.asarray(_U[0::2])
_US = jnp.asarray(_U[1::2])


def _dot(a, b, ca, cb, precision=_HI):
    return lax.dot_general(a, b, (((ca,), (cb,)), ((), ())),
                           preferred_element_type=jnp.float32,
                           precision=precision)


def _body(cls_ref, dst_ref, src_ref, ud_ref, us_ref, out_ref):
    xd = dst_ref[0]
    xs = src_ref[0]
    ud = ud_ref[...]
    us = us_ref[...]

    coef = _dot(ud, xd, 0, 0) + _dot(us, xs, 0, 0)
    hfd = xd - _dot(ud, coef, 1, 0)
    hfs = xs - _dot(us, coef, 1, 0)
    ed = jnp.sqrt(jnp.sum(hfd * hfd, axis=1))
    es = jnp.sqrt(jnp.sum(hfs * hfs, axis=1))
    pmin = jnp.minimum(jnp.min(ed), jnp.min(es))
    pmax = jnp.maximum(jnp.max(ed), jnp.max(es))
    inv = 1.0 / (pmax - pmin + 1e-6)
    phid = (ed - pmin) * inv
    phis = (es - pmin) * inv

    nd = xd / jnp.maximum(jnp.sqrt(jnp.sum(xd * xd, axis=1)), 1e-12)[:, None]
    ns = xs / jnp.maximum(jnp.sqrt(jnp.sum(xs * xs, axis=1)), 1e-12)[:, None]
    nt = 4
    ts = _NSRC // nt
    jcol_t = lax.broadcasted_iota(jnp.int32, (ts, _ND), 1)
    nm_parts, idx_parts = [], []
    for t in range(nt):
        sl = slice(t * ts, (t + 1) * ts)
        sim_t = _dot(ns[sl], nd, 1, 1, precision=None)
        adj_t = sim_t * (1.0 - _ALPHA *
                         jnp.maximum(phis[sl][:, None], phid[None, :]))
        nm_t = jnp.max(adj_t, axis=1)
        idx_t = jnp.min(jnp.where(adj_t == nm_t[:, None], jcol_t, _ND), axis=1)
        nm_parts.append(nm_t)
        idx_parts.append(idx_t)
    node_max = jnp.concatenate(nm_parts)
    node_idx = jnp.concatenate(idx_parts)

    rank = jnp.zeros((_NSRC,), jnp.float32)
    irow_t = lax.broadcasted_iota(jnp.int32, (_NSRC, ts), 0)
    for t in range(nt):
        sl = slice(t * ts, (t + 1) * ts)
        vj = node_max[sl][None, :]
        vi = node_max[:, None]
        jc = t * ts + lax.broadcasted_iota(jnp.int32, (_NSRC, ts), 1)
        better = (vj > vi) | ((vj == vi) & (jc < irow_t))
        rank = rank + jnp.sum(better.astype(jnp.float32), axis=1)
    merged = rank < (_R - 0.5)
    mf = merged.astype(jnp.float32)

    addv = jnp.zeros((_ND, _D), jnp.float32)
    cnt = jnp.zeros((_ND,), jnp.float32)
    for t in range(nt):
        sl = slice(t * ts, (t + 1) * ts)
        sel_t = jnp.where(node_idx[sl][:, None] == jcol_t, mf[sl][:, None], 0.0)
        addv = addv + _dot(sel_t, xs[sl], 0, 0)
        cnt = cnt + jnp.sum(sel_t, axis=0)
    dst_out = (xd + addv) / (1.0 + cnt)[:, None]

    keep = 1.0 - mf
    lower_t = lax.broadcasted_iota(jnp.int32, (_NSRC, ts), 1)
    pos = jnp.zeros((_NSRC,), jnp.float32)
    for t in range(nt):
        sl = slice(t * ts, (t + 1) * ts)
        contrib = jnp.where((t * ts + lower_t) < irow_t, keep[sl][None, :], 0.0)
        pos = pos + jnp.sum(contrib, axis=1)
    posi = pos.astype(jnp.int32)
    unm = jnp.zeros((_NUNM, _D), jnp.float32)
    prow_t = lax.broadcasted_iota(jnp.int32, (_NUNM, ts), 0)
    for t in range(nt):
        sl = slice(t * ts, (t + 1) * ts)
        gat_t = jnp.where((prow_t == posi[sl][None, :]) &
                          (keep[sl][None, :] > 0.5), 1.0, 0.0)
        unm = unm + _dot(gat_t, xs[sl], 1, 0)

    out_ref[0] = jnp.concatenate([cls_ref[0], dst_out, unm], axis=0)


@functools.partial(jax.jit, static_argnums=())
def kernel(tokens):
    b = tokens.shape[0]
    cls_tok = tokens[:, :1]
    dst = tokens[:, 1::2]
    src = tokens[:, 2::2]
    return pl.pallas_call(
        _body,
        grid=(b,),
        in_specs=[
            pl.BlockSpec((1, 1, _D), lambda i: (i, 0, 0)),
            pl.BlockSpec((1, _ND, _D), lambda i: (i, 0, 0)),
            pl.BlockSpec((1, _NSRC, _D), lambda i: (i, 0, 0)),
            pl.BlockSpec((_ND, 16), lambda i: (0, 0)),
            pl.BlockSpec((_NSRC, 16), lambda i: (0, 0)),
        ],
        out_specs=pl.BlockSpec((1, _NOUT, _D), lambda i: (i, 0, 0)),
        out_shape=jax.ShapeDtypeStruct((b, _NOUT, _D), jnp.float32),
    )(cls_tok, dst, src, _UD, _US)


# speedup vs baseline: 6.7805x; 1.0004x over previous
import functools

import jax
import jax.numpy as jnp
import numpy as np
from jax import lax
from jax.experimental import pallas as pl

_GRID = 32
_D = 96
_N_S = _GRID * _GRID
_ND = _N_S // 2
_NSRC = _N_S // 2
_KEEP_RATE = 0.7
_ALPHA = 0.7
_R = int(_N_S * (1.0 - _KEEP_RATE))
_NUNM = _NSRC - _R
_NOUT = 1 + _ND + _NUNM
_HI = lax.Precision.HIGHEST


def _build_low_basis():
    h = w = _GRID
    y, x = np.meshgrid(np.arange(h), np.arange(w), indexing="ij")
    cols = [np.full((h, w), 1.0 / np.sqrt(h * w))]
    for dy, dx in [(0, 1), (0, 2), (1, 0), (2, 0), (1, 1), (1, -1)]:
        ph = 2.0 * np.pi * (dy * y + dx * x) / h
        cols.append(np.sqrt(2.0 / (h * w)) * np.cos(ph))
        cols.append(np.sqrt(2.0 / (h * w)) * np.sin(ph))
    u = np.stack([c.reshape(-1) for c in cols], axis=1)
    u = np.concatenate([u, np.zeros((h * w, 3))], axis=1)
    return u.astype(np.float32)


_U = _build_low_basis()
_UD = np.ascontiguousarray(_U[0::2])
_US = np.ascontiguousarray(_U[1::2])


def _dot(a, b, ca, cb, precision=_HI):
    return lax.dot_general(a, b, (((ca,), (cb,)), ((), ())),
                           preferred_element_type=jnp.float32,
                           precision=precision)


def _body(cls_ref, ds_ref, ud_ref, us_ref, out_ref):
    cls_row = cls_ref[0]
    xd = ds_ref[0, 0]
    xs = ds_ref[0, 1]
    ud = ud_ref[...]
    us = us_ref[...]

    coef = _dot(ud, xd, 0, 0) + _dot(us, xs, 0, 0)
    hfd = xd - _dot(ud, coef, 1, 0)
    hfs = xs - _dot(us, coef, 1, 0)
    ed = jnp.sqrt(jnp.sum(hfd * hfd, axis=1))
    es = jnp.sqrt(jnp.sum(hfs * hfs, axis=1))
    pmin = jnp.minimum(jnp.min(ed), jnp.min(es))
    pmax = jnp.maximum(jnp.max(ed), jnp.max(es))
    inv = 1.0 / (pmax - pmin + 1e-6)
    phid = (ed - pmin) * inv
    phis = (es - pmin) * inv

    nd = xd / jnp.maximum(jnp.sqrt(jnp.sum(xd * xd, axis=1)), 1e-12)[:, None]
    ns = xs / jnp.maximum(jnp.sqrt(jnp.sum(xs * xs, axis=1)), 1e-12)[:, None]
    nt = 4
    ts = _NSRC // nt
    jcol_t = lax.broadcasted_iota(jnp.int32, (ts, _ND), 1)
    nm_parts, idx_parts = [], []
    for t in range(nt):
        sl = slice(t * ts, (t + 1) * ts)
        sim_t = _dot(ns[sl], nd, 1, 1, precision=None)
        adj_t = sim_t * (1.0 - _ALPHA *
                         jnp.maximum(phis[sl][:, None], phid[None, :]))
        nm_t = jnp.max(adj_t, axis=1)
        idx_t = jnp.min(jnp.where(adj_t == nm_t[:, None], jcol_t, _ND), axis=1)
        nm_parts.append(nm_t)
        idx_parts.append(idx_t)
    node_max = jnp.concatenate(nm_parts)
    node_idx = jnp.concatenate(idx_parts)

    rank = jnp.zeros((_NSRC,), jnp.float32)
    irow_t = lax.broadcasted_iota(jnp.int32, (_NSRC, ts), 0)
    for t in range(nt):
        sl = slice(t * ts, (t + 1) * ts)
        vj = node_max[sl][None, :]
        vi = node_max[:, None]
        jc = t * ts + lax.broadcasted_iota(jnp.int32, (_NSRC, ts), 1)
        better = (vj > vi) | ((vj == vi) & (jc < irow_t))
        rank = rank + jnp.sum(better.astype(jnp.float32), axis=1)
    merged = rank < (_R - 0.5)
    mf = merged.astype(jnp.float32)

    addv = jnp.zeros((_ND, _D), jnp.float32)
    cnt = jnp.zeros((_ND,), jnp.float32)
    for t in range(nt):
        sl = slice(t * ts, (t + 1) * ts)
        sel_t = jnp.where(node_idx[sl][:, None] == jcol_t, mf[sl][:, None], 0.0)
        addv = addv + _dot(sel_t, xs[sl], 0, 0)
        cnt = cnt + jnp.sum(sel_t, axis=0)
    dst_out = (xd + addv) / (1.0 + cnt)[:, None]

    keep = 1.0 - mf
    lower_t = lax.broadcasted_iota(jnp.int32, (_NSRC, ts), 1)
    pos = jnp.zeros((_NSRC,), jnp.float32)
    for t in range(nt):
        sl = slice(t * ts, (t + 1) * ts)
        contrib = jnp.where((t * ts + lower_t) < irow_t, keep[sl][None, :], 0.0)
        pos = pos + jnp.sum(contrib, axis=1)
    posi = pos.astype(jnp.int32)
    unm = jnp.zeros((_NUNM, _D), jnp.float32)
    prow_t = lax.broadcasted_iota(jnp.int32, (_NUNM, ts), 0)
    for t in range(nt):
        sl = slice(t * ts, (t + 1) * ts)
        gat_t = jnp.where((prow_t == posi[sl][None, :]) &
                          (keep[sl][None, :] > 0.5), 1.0, 0.0)
        unm = unm + _dot(gat_t, xs[sl], 1, 0)

    out_ref[0] = jnp.concatenate([cls_row, dst_out, unm], axis=0)


@functools.partial(jax.jit, static_argnums=())
def kernel(tokens):
    b = tokens.shape[0]
    cls_tok = tokens[:, :1]
    ds = tokens[:, 1:].reshape(b, _ND, 2, _D).transpose(0, 2, 1, 3)
    return pl.pallas_call(
        _body,
        grid=(b,),
        in_specs=[
            pl.BlockSpec((1, 1, _D), lambda i: (i, 0, 0)),
            pl.BlockSpec((1, 2, _ND, _D), lambda i: (i, 0, 0, 0)),
            pl.BlockSpec((_ND, 16), lambda i: (0, 0)),
            pl.BlockSpec((_NSRC, 16), lambda i: (0, 0)),
        ],
        out_specs=pl.BlockSpec((1, _NOUT, _D), lambda i: (i, 0, 0)),
        out_shape=jax.ShapeDtypeStruct((b, _NOUT, _D), jnp.float32),
    )(cls_tok, ds, _UD, _US)


# speedup vs baseline: 7.4667x; 1.1012x over previous
import functools

import jax
import jax.numpy as jnp
import numpy as np
from jax import lax
from jax.experimental import pallas as pl

_GRID = 32
_D = 96
_N_S = _GRID * _GRID
_ND = _N_S // 2
_NSRC = _N_S // 2
_KEEP_RATE = 0.7
_ALPHA = 0.7
_R = int(_N_S * (1.0 - _KEEP_RATE))
_NUNM = _NSRC - _R
_NOUT = 1 + _ND + _NUNM
_HI = lax.Precision.HIGHEST
_BS = 2


def _build_low_basis():
    h = w = _GRID
    y, x = np.meshgrid(np.arange(h), np.arange(w), indexing="ij")
    cols = [np.full((h, w), 1.0 / np.sqrt(h * w))]
    for dy, dx in [(0, 1), (0, 2), (1, 0), (2, 0), (1, 1), (1, -1)]:
        ph = 2.0 * np.pi * (dy * y + dx * x) / h
        cols.append(np.sqrt(2.0 / (h * w)) * np.cos(ph))
        cols.append(np.sqrt(2.0 / (h * w)) * np.sin(ph))
    u = np.stack([c.reshape(-1) for c in cols], axis=1)
    u = np.concatenate([u, np.zeros((h * w, 3))], axis=1)
    return u.astype(np.float32)


_U = _build_low_basis()
_UD = np.ascontiguousarray(_U[0::2])
_US = np.ascontiguousarray(_U[1::2])


def _dot(a, b, ca, cb, precision=_HI):
    return lax.dot_general(a, b, (((ca,), (cb,)), ((), ())),
                           preferred_element_type=jnp.float32,
                           precision=precision)


def _one_sample(cls_row, xd, xs, ud, us):

    coef = _dot(ud, xd, 0, 0) + _dot(us, xs, 0, 0)
    hfd = xd - _dot(ud, coef, 1, 0)
    hfs = xs - _dot(us, coef, 1, 0)
    ed = jnp.sqrt(jnp.sum(hfd * hfd, axis=1, keepdims=True))
    es = jnp.sqrt(jnp.sum(hfs * hfs, axis=1, keepdims=True))
    pmin = jnp.minimum(jnp.min(ed), jnp.min(es))
    pmax = jnp.maximum(jnp.max(ed), jnp.max(es))
    inv = 1.0 / (pmax - pmin + 1e-6)
    phid_c = (ed - pmin) * inv
    phis_c = (es - pmin) * inv
    phid_r = phid_c.T

    nd = xd / jnp.maximum(jnp.sqrt(jnp.sum(xd * xd, axis=1, keepdims=True)),
                          1e-12)
    ns = xs / jnp.maximum(jnp.sqrt(jnp.sum(xs * xs, axis=1, keepdims=True)),
                          1e-12)
    nt = 4
    ts = _NSRC // nt
    jcol_t = lax.broadcasted_iota(jnp.int32, (ts, _ND), 1)
    nm_parts, idx_parts = [], []
    for t in range(nt):
        sl = slice(t * ts, (t + 1) * ts)
        sim_t = _dot(ns[sl], nd, 1, 1, precision=None)
        adj_t = sim_t * (1.0 - _ALPHA * jnp.maximum(phis_c[sl], phid_r))
        nm_t = jnp.max(adj_t, axis=1, keepdims=True)
        idx_t = jnp.min(jnp.where(adj_t == nm_t, jcol_t, _ND), axis=1,
                        keepdims=True)
        nm_parts.append(nm_t)
        idx_parts.append(idx_t)
    nm_c = jnp.concatenate(nm_parts, axis=0)
    idx_c = jnp.concatenate(idx_parts, axis=0)
    nm_r = nm_c.T

    irow_t = lax.broadcasted_iota(jnp.int32, (_NSRC, ts), 0)
    jcol_s = lax.broadcasted_iota(jnp.int32, (_NSRC, ts), 1)
    rank = jnp.zeros((_NSRC, 1), jnp.float32)
    for t in range(nt):
        sl = slice(t * ts, (t + 1) * ts)
        vj = nm_r[:, sl]
        better = (vj > nm_c) | ((vj == nm_c) & ((t * ts + jcol_s) < irow_t))
        rank = rank + jnp.sum(better.astype(jnp.float32), axis=1,
                              keepdims=True)
    mf_c = (rank < (_R - 0.5)).astype(jnp.float32)
    mf_r = mf_c.T
    keep_r = 1.0 - mf_r

    addv = jnp.zeros((_ND, _D), jnp.float32)
    cnt_r = jnp.zeros((1, _ND), jnp.float32)
    for t in range(nt):
        sl = slice(t * ts, (t + 1) * ts)
        sel_t = jnp.where(idx_c[sl] == jcol_t, mf_c[sl], 0.0)
        addv = addv + _dot(sel_t, xs[sl], 0, 0)
        cnt_r = cnt_r + jnp.sum(sel_t, axis=0, keepdims=True)
    dst_out = (xd + addv) / (1.0 + cnt_r.T)

    pos_c = jnp.zeros((_NSRC, 1), jnp.float32)
    for t in range(nt):
        sl = slice(t * ts, (t + 1) * ts)
        contrib = jnp.where((t * ts + jcol_s) < irow_t, keep_r[:, sl], 0.0)
        pos_c = pos_c + jnp.sum(contrib, axis=1, keepdims=True)
    posi_r = pos_c.astype(jnp.int32).T
    unm = jnp.zeros((_NUNM, _D), jnp.float32)
    prow_t = lax.broadcasted_iota(jnp.int32, (_NUNM, ts), 0)
    for t in range(nt):
        sl = slice(t * ts, (t + 1) * ts)
        gat_t = jnp.where((prow_t == posi_r[:, sl]) & (keep_r[:, sl] > 0.5),
                          1.0, 0.0)
        unm = unm + _dot(gat_t, xs[sl], 1, 0)

    return jnp.concatenate([cls_row, dst_out, unm], axis=0)


def _body(cls_ref, ds_ref, ud_ref, us_ref, out_ref):
    ud = ud_ref[...]
    us = us_ref[...]
    for s in range(_BS):
        out_ref[s] = _one_sample(cls_ref[s], ds_ref[s, 0], ds_ref[s, 1],
                                 ud, us)


@functools.partial(jax.jit, static_argnums=())
def kernel(tokens):
    b = tokens.shape[0]
    cls_tok = tokens[:, :1]
    ds = tokens[:, 1:].reshape(b, _ND, 2, _D).transpose(0, 2, 1, 3)
    return pl.pallas_call(
        _body,
        grid=(b // _BS,),
        in_specs=[
            pl.BlockSpec((_BS, 1, _D), lambda i: (i, 0, 0)),
            pl.BlockSpec((_BS, 2, _ND, _D), lambda i: (i, 0, 0, 0)),
            pl.BlockSpec((_ND, 16), lambda i: (0, 0)),
            pl.BlockSpec((_NSRC, 16), lambda i: (0, 0)),
        ],
        out_specs=pl.BlockSpec((_BS, _NOUT, _D), lambda i: (i, 0, 0)),
        out_shape=jax.ShapeDtypeStruct((b, _NOUT, _D), jnp.float32),
    )(cls_tok, ds, _UD, _US)
